# baseline (speedup 1.0000x reference)
import math

import jax
import jax.numpy as jnp
from jax.experimental import pallas as pl
from jax.experimental.pallas import tpu as pltpu

B, H, L, Dh = 1, 12, 2048, 64
K, M = 6, 2
L_train, L_max = 2048, 1000000
ramp_lambda = 0.2
tau = 64.0
c_scale = 0.01
width_min, width_max = 32.0, 256.0
delta_star_max = L_train - 1

BI = 1024
BJ = 1024

_LG10_WMIN = math.log10(2.0 * math.pi / L_max)
_LG10_WMAX = math.log10(2.0 * math.pi / L_train)
_DLG = (_LG10_WMAX - _LG10_WMIN) / (K - 1)
_LN10 = math.log(10.0)
_HALFPI = 0.5 * math.pi
_SQH_L2E = math.sqrt(0.5 * math.log2(math.e))
_RAMP_L2 = -ramp_lambda * math.log(2.0)


def _omega_from_k(kf):
    return jnp.exp(_LN10 * (_LG10_WMIN + kf * _DLG))


def _bias_kernel(qt_ref, wt_ref, out_ref, coef_ref, scal_ref, tab_ref,
                 dbase_ref, lr_ref):
    h = pl.program_id(0)
    ib = pl.program_id(1)
    jb = pl.program_id(2)

    i0 = (ib * BI).astype(jnp.float32)
    j0 = (jb * BJ).astype(jnp.float32)

    @pl.when(jnp.logical_and(h == 0, jnp.logical_and(ib == 0, jb == 0)))
    def _build_tab():
        r = jax.lax.broadcasted_iota(jnp.int32, (16, 1), 0)
        kr = (r % K).astype(jnp.float32)
        wr = jnp.where(r < 2 * K, _omega_from_k(kr), 0.0)
        ph = jnp.where(r < K, 0.0, _HALFPI)
        jcol = jax.lax.broadcasted_iota(
            jnp.int32, (16, L), 1).astype(jnp.float32)
        tab_ref[...] = jnp.cos(wr * jcol - ph)

        dbase_ref[...] = (
            jax.lax.broadcasted_iota(jnp.int32, (BI, BJ), 0)
            - jax.lax.broadcasted_iota(jnp.int32, (BI, BJ), 1)
        ).astype(jnp.float32)

    @pl.when(jnp.logical_and(h == 0, jb == 0))
    def _build_lr():
        lr_ref[ib] = jnp.log2(dbase_ref[...] + (tau + i0))

    @pl.when(jb == 0)
    def _prologue():
        qtb = qt_ref[0, 0]
        projT = jax.lax.dot_general(
            wt_ref[...], qtb, (((1,), (0,)), ((), ())),
            preferred_element_type=jnp.float32)
        a = projT[0:12, :] * c_scale
        bcoef = projT[12:24, :] * c_scale
        mu = jax.nn.sigmoid(projT[24:26, :]) * float(delta_star_max)
        sig = width_min + jax.nn.sigmoid(projT[26:28, :]) * (width_max - width_min)
        pr = projT[28:30, :]
        pmx = jnp.maximum(pr[0:1, :], pr[1:2, :])
        pe = jnp.exp(pr - pmx)
        pm = pe / (pe[0:1, :] + pe[1:2, :])
        gate = jax.nn.softplus(projT[30:31, :])

        rr = jax.lax.broadcasted_iota(jnp.int32, (24, 1), 0)
        omg = _omega_from_k((rr % K).astype(jnp.float32))
        ph = jnp.where(rr < 2 * K, 0.0, _HALFPI)
        irow = i0 + jax.lax.broadcasted_iota(
            jnp.int32, (1, BI), 1).astype(jnp.float32)
        cs = jnp.cos(omg * irow - ph)
        ci = cs[0:12, :]
        si = cs[12:24, :]
        gpm = gate * jnp.repeat(pm, K, axis=0)
        ca = gpm * (a * ci + bcoef * si)
        cb = gpm * (a * si - bcoef * ci)

        z4 = jnp.zeros((4, BI), jnp.float32)
        coef_ref[0:BI, :] = jnp.concatenate(
            [ca[0:K, :], cb[0:K, :], z4], axis=0).T
        coef_ref[BI:2 * BI, :] = jnp.concatenate(
            [ca[K:2 * K, :], cb[K:2 * K, :], z4], axis=0).T

        rat = mu / sig
        env0m = pm * jnp.exp(-0.5 * rat * rat)
        suma0 = jnp.sum(a[0:K, :], axis=0, keepdims=True)
        suma1 = jnp.sum(a[K:2 * K, :], axis=0, keepdims=True)
        b0 = env0m[0:1, :] * suma0 + env0m[1:2, :] * suma1

        isig = _SQH_L2E / sig
        nis2 = -(isig * isig)
        g1 = gate * _RAMP_L2
        g2 = gate * (-6.0 * _RAMP_L2) - gate * b0
        scal_ref[...] = jnp.concatenate(
            [mu, nis2, g1, g2, jnp.zeros((2, BI), jnp.float32)], axis=0).T

    zero_tile = j0 > i0 + float(BI - 1)
    interior = j0 + float(BJ - 1) <= i0

    @pl.when(zero_tile)
    def _zeros():
        out_ref[0, 0] = jnp.zeros((BI, BJ), jnp.float32)

    def _body(masked):
        tabblk = tab_ref[:, pl.ds(jb * BJ, BJ)]
        F = jax.lax.dot_general(
            coef_ref[...], tabblk, (((1,), (0,)), ((), ())),
            preferred_element_type=jnp.float32)
        four0 = F[0:BI, :]
        four1 = F[BI:2 * BI, :]

        shift = i0 - j0
        sc = scal_ref[...]
        m0s = sc[:, 0:1] - shift
        m1s = sc[:, 1:2] - shift
        nis20 = sc[:, 2:3]
        nis21 = sc[:, 3:4]
        g1 = sc[:, 4:5]
        g2 = sc[:, 5:6]

        dbase = dbase_ref[...]
        u0 = dbase - m0s
        e0 = jnp.exp2(u0 * (u0 * nis20))
        u1 = dbase - m1s
        e1 = jnp.exp2(u1 * (u1 * nis21))

        lr = lr_ref[ib - (BJ // BI) * jb]

        res = four0 * e0 + four1 * e1 + g1 * lr + g2
        if masked:
            res = jnp.where(dbase >= (j0 - i0), res, 0.0)
        out_ref[0, 0] = res

    @pl.when(interior)
    def _interior():
        _body(False)

    @pl.when(jnp.logical_and(jnp.logical_not(zero_tile),
                             jnp.logical_not(interior)))
    def _edge():
        _body(True)


def kernel(q, W_a, W_b, W_c, W_w, W_pi, W_g):
    W_all_t = jnp.concatenate(
        [W_a, W_b, W_c, W_w, W_pi, W_g[:, None],
         jnp.zeros((Dh, 1), jnp.float32)], axis=1).T
    qt = jnp.swapaxes(q, -1, -2)

    grid = (H, L // BI, L // BJ)
    out = pl.pallas_call(
        _bias_kernel,
        grid=grid,
        in_specs=[
            pl.BlockSpec((1, 1, Dh, BI), lambda h, ib, jb: (0, h, 0, ib)),
            pl.BlockSpec((32, Dh), lambda h, ib, jb: (0, 0)),
        ],
        out_specs=pl.BlockSpec((1, 1, BI, BJ), lambda h, ib, jb: (0, h, ib, jb)),
        out_shape=jax.ShapeDtypeStruct((B, H, L, L), jnp.float32),
        scratch_shapes=[
            pltpu.VMEM((2 * BI, 16), jnp.float32),
            pltpu.VMEM((BI, 8), jnp.float32),
            pltpu.VMEM((16, L), jnp.float32),
            pltpu.VMEM((BI, BJ), jnp.float32),
            pltpu.VMEM((L // BI, BI, BJ), jnp.float32),
        ],
    )(qt, W_all_t)
    return out


# speedup vs baseline: 1.2680x; 1.2680x over previous
import math

import jax
import jax.numpy as jnp
from jax.experimental import pallas as pl
from jax.experimental.pallas import tpu as pltpu

B, H, L, Dh = 1, 12, 2048, 64
K, M = 6, 2
L_train, L_max = 2048, 1000000
ramp_lambda = 0.2
tau = 64.0
c_scale = 0.01
width_min, width_max = 32.0, 256.0
delta_star_max = L_train - 1

BI = 512
BJ = 2048

_LG10_WMIN = math.log10(2.0 * math.pi / L_max)
_LG10_WMAX = math.log10(2.0 * math.pi / L_train)
_DLG = (_LG10_WMAX - _LG10_WMIN) / (K - 1)
_LN10 = math.log(10.0)
_HALFPI = 0.5 * math.pi
_SQH_L2E = math.sqrt(0.5 * math.log2(math.e))
_RAMP_L2 = -ramp_lambda * math.log(2.0)


def _omega_from_k(kf):
    return jnp.exp(_LN10 * (_LG10_WMIN + kf * _DLG))


def _bias_kernel(qt_ref, wt_ref, out_ref, coef_ref, scal_ref, tab_ref,
                 dbase_ref, lr_ref):
    h = pl.program_id(0)
    ib = pl.program_id(1)
    jb = pl.program_id(2)

    i0 = (ib * BI).astype(jnp.float32)
    j0 = (jb * BJ).astype(jnp.float32)

    @pl.when(jnp.logical_and(h == 0, jnp.logical_and(ib == 0, jb == 0)))
    def _build_tab():
        r = jax.lax.broadcasted_iota(jnp.int32, (16, 1), 0)
        kr = (r % K).astype(jnp.float32)
        wr = jnp.where(r < 2 * K, _omega_from_k(kr), 0.0)
        ph = jnp.where(r < K, 0.0, _HALFPI)
        jcol = jax.lax.broadcasted_iota(
            jnp.int32, (16, L), 1).astype(jnp.float32)
        tab_ref[...] = jnp.cos(wr * jcol - ph)

        dbase_ref[...] = (
            jax.lax.broadcasted_iota(jnp.int32, (BI, BJ), 0)
            - jax.lax.broadcasted_iota(jnp.int32, (BI, BJ), 1)
        ).astype(jnp.float32)

    @pl.when(jnp.logical_and(h == 0, jb == 0))
    def _build_lr():
        lr_ref[ib] = jnp.log2(dbase_ref[...] + (tau + i0))

    @pl.when(jb == 0)
    def _prologue():
        qtb = qt_ref[0, 0]
        projT = jax.lax.dot_general(
            wt_ref[...], qtb, (((1,), (0,)), ((), ())),
            preferred_element_type=jnp.float32)
        a = projT[0:12, :] * c_scale
        bcoef = projT[12:24, :] * c_scale
        mu = jax.nn.sigmoid(projT[24:26, :]) * float(delta_star_max)
        sig = width_min + jax.nn.sigmoid(projT[26:28, :]) * (width_max - width_min)
        pr = projT[28:30, :]
        pmx = jnp.maximum(pr[0:1, :], pr[1:2, :])
        pe = jnp.exp(pr - pmx)
        pm = pe / (pe[0:1, :] + pe[1:2, :])
        gate = jax.nn.softplus(projT[30:31, :])

        rr = jax.lax.broadcasted_iota(jnp.int32, (24, 1), 0)
        omg = _omega_from_k((rr % K).astype(jnp.float32))
        ph = jnp.where(rr < 2 * K, 0.0, _HALFPI)
        irow = i0 + jax.lax.broadcasted_iota(
            jnp.int32, (1, BI), 1).astype(jnp.float32)
        cs = jnp.cos(omg * irow - ph)
        ci = cs[0:12, :]
        si = cs[12:24, :]
        gpm = gate * jnp.repeat(pm, K, axis=0)
        ca = gpm * (a * ci + bcoef * si)
        cb = gpm * (a * si - bcoef * ci)

        z4 = jnp.zeros((4, BI), jnp.float32)
        coef_ref[0:BI, :] = jnp.concatenate(
            [ca[0:K, :], cb[0:K, :], z4], axis=0).T
        coef_ref[BI:2 * BI, :] = jnp.concatenate(
            [ca[K:2 * K, :], cb[K:2 * K, :], z4], axis=0).T

        rat = mu / sig
        env0m = pm * jnp.exp(-0.5 * rat * rat)
        suma0 = jnp.sum(a[0:K, :], axis=0, keepdims=True)
        suma1 = jnp.sum(a[K:2 * K, :], axis=0, keepdims=True)
        b0 = env0m[0:1, :] * suma0 + env0m[1:2, :] * suma1

        isig = _SQH_L2E / sig
        nis2 = -(isig * isig)
        g1 = gate * _RAMP_L2
        g2 = gate * (-6.0 * _RAMP_L2) - gate * b0
        scal_ref[...] = jnp.concatenate(
            [mu, nis2, g1, g2, jnp.zeros((2, BI), jnp.float32)], axis=0).T

    def _body(width):
        tabblk = tab_ref[:, 0:width]
        F = jax.lax.dot_general(
            coef_ref[...], tabblk, (((1,), (0,)), ((), ())),
            preferred_element_type=jnp.float32)
        four0 = F[0:BI, :]
        four1 = F[BI:2 * BI, :]

        sc = scal_ref[...]
        m0s = sc[:, 0:1] - i0
        m1s = sc[:, 1:2] - i0
        nis20 = sc[:, 2:3]
        nis21 = sc[:, 3:4]
        g1 = sc[:, 4:5]
        g2 = sc[:, 5:6]

        dbase = dbase_ref[:, 0:width]
        u0 = dbase - m0s
        e0 = jnp.exp2(u0 * (u0 * nis20))
        u1 = dbase - m1s
        e1 = jnp.exp2(u1 * (u1 * nis21))

        lr = lr_ref[ib, :, 0:width]

        res = four0 * e0 + four1 * e1 + g1 * lr + g2
        res = jnp.where(dbase >= -i0, res, 0.0)
        out_ref[0, 0, :, 0:width] = res
        if width < BJ:
            out_ref[0, 0, :, width:BJ] = jnp.zeros(
                (BI, BJ - width), jnp.float32)

    for _ibv in range(L // BI):
        @pl.when(ib == _ibv)
        def _run(_ibv=_ibv):
            _body((_ibv + 1) * BI)


def kernel(q, W_a, W_b, W_c, W_w, W_pi, W_g):
    W_all_t = jnp.concatenate(
        [W_a, W_b, W_c, W_w, W_pi, W_g[:, None],
         jnp.zeros((Dh, 1), jnp.float32)], axis=1).T
    qt = jnp.swapaxes(q, -1, -2)

    grid = (H, L // BI, L // BJ)
    out = pl.pallas_call(
        _bias_kernel,
        grid=grid,
        in_specs=[
            pl.BlockSpec((1, 1, Dh, BI), lambda h, ib, jb: (0, h, 0, ib)),
            pl.BlockSpec((32, Dh), lambda h, ib, jb: (0, 0)),
        ],
        out_specs=pl.BlockSpec((1, 1, BI, BJ), lambda h, ib, jb: (0, h, ib, jb)),
        out_shape=jax.ShapeDtypeStruct((B, H, L, L), jnp.float32),
        scratch_shapes=[
            pltpu.VMEM((2 * BI, 16), jnp.float32),
            pltpu.VMEM((BI, 8), jnp.float32),
            pltpu.VMEM((16, L), jnp.float32),
            pltpu.VMEM((BI, BJ), jnp.float32),
            pltpu.VMEM((L // BI, BI, BJ), jnp.float32),
        ],
    )(qt, W_all_t)
    return out
